# baseline (speedup 1.0000x reference)
import functools

import jax
import jax.numpy as jnp
from jax import lax
from jax.experimental import pallas as pl
from jax.experimental.pallas import tpu as pltpu
from jax.experimental.pallas import tpu_sc as plsc

D = 128
NC = 2
NS = 16
NW = NC * NS
CHUNK = 64
GSZ = 8
NPAD = 10240


def _tc_gru_xw(x, w0, w_ih, w_hh, b_ih2, b_hh2):
    def body(x_ref, w0_ref, wih_ref, whh_ref, bih_ref, bhh_ref, out_ref):
        w = w0_ref[...]
        gi = lax.dot_general(w, wih_ref[...], (((1,), (1,)), ((), ())),
                             preferred_element_type=jnp.float32) + bih_ref[...]
        gh = lax.dot_general(w, whh_ref[...], (((1,), (1,)), ((), ())),
                             preferred_element_type=jnp.float32) + bhh_ref[...]
        r = jax.nn.sigmoid(gi[:, :D] + gh[:, :D])
        z = jax.nn.sigmoid(gi[:, D:2 * D] + gh[:, D:2 * D])
        n = jnp.tanh(gi[:, 2 * D:] + r * gh[:, 2 * D:])
        wt = (1.0 - z) * n + z * w
        out_ref[...] = jnp.dot(x_ref[...], wt,
                               preferred_element_type=jnp.float32)

    return pl.pallas_call(
        body,
        out_shape=jax.ShapeDtypeStruct((x.shape[0], D), jnp.float32),
    )(x, w0, w_ih, w_hh, b_ih2, b_hh2)


def _sc_degree(dst4d):
    ngrp = dst4d.shape[1]
    spt = NPAD // NS
    mesh = plsc.VectorSubcoreMesh(core_axis_name="c", subcore_axis_name="s")

    @functools.partial(
        pl.kernel,
        out_type=jax.ShapeDtypeStruct((NC, 1, NPAD), jnp.float32),
        mesh=mesh,
        scratch_types=[
            pltpu.VMEM((GSZ, CHUNK), jnp.int32),
            pltpu.VMEM((CHUNK,), jnp.float32),
            pltpu.VMEM((spt,), jnp.float32),
            pltpu.VMEM_SHARED((NPAD,), jnp.float32),
            pltpu.SemaphoreType.DMA,
        ],
    )
    def k(dst_hbm, out_hbm, idx_v, ones_v, buf_v, acc_sh, sem):
        c = lax.axis_index("c")
        s = lax.axis_index("s")
        wid = s * NC + c

        def fill_zero(i, _):
            buf_v[pl.ds(i * 16, 16)] = jnp.zeros((16,), jnp.float32)
            return 0
        lax.fori_loop(0, spt // 16, fill_zero, 0)

        def fill_one(i, _):
            ones_v[pl.ds(i * 16, 16)] = jnp.ones((16,), jnp.float32)
            return 0
        lax.fori_loop(0, CHUNK // 16, fill_one, 0)

        pltpu.sync_copy(buf_v, acc_sh.at[pl.ds(s * spt, spt)])
        plsc.subcore_barrier()

        def grp(g, _):
            pltpu.sync_copy(dst_hbm.at[wid, g], idx_v)
            for t in range(GSZ):
                pltpu.async_copy(ones_v, acc_sh.at[idx_v.at[t]], sem, add=True)
            for t in range(GSZ):
                pltpu.make_async_copy(ones_v, acc_sh.at[idx_v.at[t]], sem).wait()
            return 0
        lax.fori_loop(0, ngrp, grp, 0)
        plsc.subcore_barrier()

        pltpu.sync_copy(acc_sh.at[pl.ds(s * spt, spt)], buf_v)
        pltpu.sync_copy(buf_v, out_hbm.at[c, 0, pl.ds(s * spt, spt)])

    return k(dst4d)


def _sc_scatter(src4d, dst4d, y, zeros_h):
    ngrp = src4d.shape[1]
    npt = NPAD // NS
    zb = zeros_h.shape[0]
    nz = npt // zb
    mesh = plsc.VectorSubcoreMesh(core_axis_name="c", subcore_axis_name="s")

    @functools.partial(
        pl.kernel,
        out_type=jax.ShapeDtypeStruct((NC, NPAD, D), jnp.float32),
        mesh=mesh,
        scratch_types=[
            pltpu.VMEM((2, GSZ, CHUNK), jnp.int32),
            pltpu.VMEM((2, GSZ, CHUNK), jnp.int32),
            pltpu.VMEM((2, CHUNK, D), jnp.float32),
            pltpu.VMEM_SHARED((NPAD, D), jnp.float32),
            pltpu.SemaphoreType.DMA,
            pltpu.SemaphoreType.DMA,
            pltpu.SemaphoreType.DMA,
            pltpu.SemaphoreType.DMA,
        ],
    )
    def k(src_hbm, dst_hbm, y_hbm, z_hbm, out_hbm,
          si_v, di_v, rows_v, acc_sh, gsem0, gsem1, ssem0, ssem1):
        c = lax.axis_index("c")
        s = lax.axis_index("s")
        wid = s * NC + c
        r0 = s * npt
        gsems = (gsem0, gsem1)
        ssems = (ssem0, ssem1)

        pltpu.sync_copy(z_hbm, rows_v.at[0])
        for i in range(nz):
            pltpu.sync_copy(rows_v.at[0], acc_sh.at[pl.ds(r0 + i * zb, zb), :])
        pltpu.sync_copy(src_hbm.at[wid, 0], si_v.at[0])
        pltpu.sync_copy(dst_hbm.at[wid, 0], di_v.at[0])
        pltpu.async_copy(y_hbm.at[si_v.at[0, 0]], rows_v.at[0], gsems[0])
        plsc.subcore_barrier()

        def wait_scatter(b, p, t):
            pltpu.make_async_copy(
                rows_v.at[b], acc_sh.at[di_v.at[p, t]], ssems[b]).wait()

        def grp(g, _):
            p = lax.rem(g, 2)
            q = 1 - p

            @pl.when(g > 0)
            def _settle():
                wait_scatter(1, p, 0)

            @pl.when(g < ngrp - 1)
            def _stage():
                pltpu.sync_copy(src_hbm.at[wid, g + 1], si_v.at[q])
                pltpu.sync_copy(dst_hbm.at[wid, g + 1], di_v.at[q])

            for t in range(GSZ):
                b = t % 2
                nb = 1 - b
                if t > 0:
                    wait_scatter(nb, p, t)
                if t < GSZ - 1:
                    nxt = si_v.at[p, t + 1]
                else:
                    nxt = si_v.at[q, 0]
                pltpu.async_copy(y_hbm.at[nxt], rows_v.at[nb], gsems[nb])
                pltpu.make_async_copy(
                    y_hbm.at[si_v.at[p, t]], rows_v.at[b], gsems[b]).wait()
                pltpu.async_copy(rows_v.at[b], acc_sh.at[di_v.at[p, t]],
                                 ssems[b], add=True)
            return 0
        lax.fori_loop(0, ngrp, grp, 0)
        wait_scatter(1, 0, 0)
        pltpu.make_async_copy(y_hbm.at[si_v.at[0, 0]], rows_v.at[0],
                              gsems[0]).wait()
        plsc.subcore_barrier()

        for i in range(nz):
            pltpu.sync_copy(acc_sh.at[pl.ds(r0 + i * zb, zb), :], rows_v.at[0])
            pltpu.sync_copy(rows_v.at[0], out_hbm.at[c, pl.ds(r0 + i * zb, zb), :])

    return k(src4d, dst4d, y, zeros_h)


def _tc_scale(xw, degp):
    n = xw.shape[0]

    def body(xw_ref, degp_ref, y_ref, dinv_ref):
        deg = degp_ref[0] + degp_ref[1] + 1.0
        dinv = lax.rsqrt(deg)
        dinv_ref[...] = dinv
        y_ref[:n] = xw_ref[...] * dinv[:n]
        y_ref[n:] = jnp.zeros((NPAD - n, D), jnp.float32)

    return pl.pallas_call(
        body,
        out_shape=(
            jax.ShapeDtypeStruct((NPAD, D), jnp.float32),
            jax.ShapeDtypeStruct((NPAD, 1), jnp.float32),
        ),
    )(xw, degp)


def _tc_final(partials, y, dinv, n):
    def body(p_ref, y_ref, dinv_ref, out_ref):
        acc = p_ref[0][:n] + p_ref[1][:n] + y_ref[:n]
        out_ref[...] = acc * dinv_ref[:n]

    return pl.pallas_call(
        body,
        out_shape=jax.ShapeDtypeStruct((n, D), jnp.float32),
    )(partials, y, dinv)


def kernel(x, edge_index, initial_weight, w_ih, w_hh, b_ih, b_hh):
    n = x.shape[0]
    e = edge_index.shape[1]
    ngrp = -(-e // (NW * GSZ * CHUNK))
    e2 = NW * ngrp * GSZ * CHUNK
    assert n < NPAD and e2 >= e

    pad = e2 - e
    padi = (jnp.arange(pad, dtype=jnp.int32) % (NPAD - n)) + n
    src4 = jnp.concatenate([edge_index[0], padi]).reshape(NW, ngrp, GSZ, CHUNK)
    dst4 = jnp.concatenate([edge_index[1], padi]).reshape(NW, ngrp, GSZ, CHUNK)
    b_ih2 = b_ih.reshape(1, 3 * D)
    b_hh2 = b_hh.reshape(1, 3 * D)
    zeros_h = jnp.zeros((CHUNK, D), jnp.float32)

    degp = _sc_degree(dst4)
    xw = _tc_gru_xw(x, initial_weight, w_ih, w_hh, b_ih2, b_hh2)
    y, dinv = _tc_scale(xw, degp.reshape(NC, NPAD, 1))
    partials = _sc_scatter(src4, dst4, y, zeros_h)
    return _tc_final(partials, y, dinv, n)


# speedup vs baseline: 1.2470x; 1.2470x over previous
import functools

import jax
import jax.numpy as jnp
from jax import lax
from jax.experimental import pallas as pl
from jax.experimental.pallas import tpu as pltpu
from jax.experimental.pallas import tpu_sc as plsc

D = 128
NC = 2
NS = 16
NW = NC * NS
CHUNK = 80
NB = 4
NPAD = 10240


def _tc_gru_xw(x, w0, w_ih, w_hh, b_ih2, b_hh2):
    def body(x_ref, w0_ref, wih_ref, whh_ref, bih_ref, bhh_ref, out_ref):
        w = w0_ref[...]
        gi = lax.dot_general(w, wih_ref[...], (((1,), (1,)), ((), ())),
                             preferred_element_type=jnp.float32) + bih_ref[...]
        gh = lax.dot_general(w, whh_ref[...], (((1,), (1,)), ((), ())),
                             preferred_element_type=jnp.float32) + bhh_ref[...]
        r = jax.nn.sigmoid(gi[:, :D] + gh[:, :D])
        z = jax.nn.sigmoid(gi[:, D:2 * D] + gh[:, D:2 * D])
        n = jnp.tanh(gi[:, 2 * D:] + r * gh[:, 2 * D:])
        wt = (1.0 - z) * n + z * w
        out_ref[...] = jnp.dot(x_ref[...], wt,
                               preferred_element_type=jnp.float32)

    return pl.pallas_call(
        body,
        out_shape=jax.ShapeDtypeStruct((x.shape[0], D), jnp.float32),
    )(x, w0, w_ih, w_hh, b_ih2, b_hh2)


def _sc_degree(dst4d):
    ngrp = dst4d.shape[1]
    spt = NPAD // NS
    mesh = plsc.VectorSubcoreMesh(core_axis_name="c", subcore_axis_name="s")

    @functools.partial(
        pl.kernel,
        out_type=jax.ShapeDtypeStruct((NC, 1, NPAD), jnp.float32),
        mesh=mesh,
        scratch_types=[
            pltpu.VMEM((2, NB, CHUNK), jnp.int32),
            pltpu.VMEM((CHUNK,), jnp.float32),
            pltpu.VMEM((spt,), jnp.float32),
            pltpu.VMEM_SHARED((NPAD,), jnp.float32),
            pltpu.SemaphoreType.DMA,
        ],
    )
    def k(dst_hbm, out_hbm, idx_v, ones_v, buf_v, acc_sh, sem):
        c = lax.axis_index("c")
        s = lax.axis_index("s")
        wid = s * NC + c

        def fill_zero(i, _):
            buf_v[pl.ds(i * 16, 16)] = jnp.zeros((16,), jnp.float32)
            return 0
        lax.fori_loop(0, spt // 16, fill_zero, 0)

        def fill_one(i, _):
            ones_v[pl.ds(i * 16, 16)] = jnp.ones((16,), jnp.float32)
            return 0
        lax.fori_loop(0, CHUNK // 16, fill_one, 0)

        pltpu.sync_copy(buf_v, acc_sh.at[pl.ds(s * spt, spt)])
        plsc.subcore_barrier()

        def grp(g, _):
            pltpu.sync_copy(dst_hbm.at[wid, pl.ds(2 * g, 2)], idx_v)
            for u in range(2):
                for t in range(NB):
                    pltpu.async_copy(ones_v, acc_sh.at[idx_v.at[u, t]], sem,
                                     add=True)
            for u in range(2):
                for t in range(NB):
                    pltpu.make_async_copy(ones_v, acc_sh.at[idx_v.at[u, t]],
                                          sem).wait()
            return 0
        lax.fori_loop(0, ngrp // 2, grp, 0)
        plsc.subcore_barrier()

        pltpu.sync_copy(acc_sh.at[pl.ds(s * spt, spt)], buf_v)
        pltpu.sync_copy(buf_v, out_hbm.at[c, 0, pl.ds(s * spt, spt)])

    return k(dst4d)


def _sc_scatter(src4d, dst4d, y, zeros_h):
    ngrp = src4d.shape[1]
    npt = NPAD // NS
    mesh = plsc.VectorSubcoreMesh(core_axis_name="c", subcore_axis_name="s")

    @functools.partial(
        pl.kernel,
        out_type=jax.ShapeDtypeStruct((NC, NPAD, D), jnp.float32),
        mesh=mesh,
        scratch_types=[
            pltpu.VMEM((3, NB, CHUNK), jnp.int32),
            pltpu.VMEM((3, NB, CHUNK), jnp.int32),
            pltpu.VMEM((NB, CHUNK, D), jnp.float32),
            pltpu.VMEM_SHARED((NPAD, D), jnp.float32),
            [pltpu.SemaphoreType.DMA] * NB,
            [pltpu.SemaphoreType.DMA] * NB,
            [pltpu.SemaphoreType.DMA] * 2,
        ],
    )
    def k(src_hbm, dst_hbm, y_hbm, z_hbm, out_hbm,
          si_v, di_v, rows_v, acc_sh, gsems, ssems, stsems):
        c = lax.axis_index("c")
        s = lax.axis_index("s")
        wid = s * NC + c
        r0 = s * npt

        def issue_stage(m, sem):
            b = lax.rem(m, 3)
            pltpu.async_copy(src_hbm.at[wid, m], si_v.at[b], sem)
            pltpu.async_copy(dst_hbm.at[wid, m], di_v.at[b], sem)

        def wait_stage(bg, sem):
            pltpu.make_async_copy(src_hbm.at[wid, 0], si_v.at[bg], sem).wait()
            pltpu.make_async_copy(dst_hbm.at[wid, 0], di_v.at[bg], sem).wait()

        def wait_gather(b):
            pltpu.make_async_copy(
                y_hbm.at[si_v.at[0, 0]], rows_v.at[b], gsems[b]).wait()

        def wait_scatter(b):
            pltpu.make_async_copy(
                rows_v.at[b], acc_sh.at[di_v.at[0, 0]], ssems[b]).wait()

        pltpu.sync_copy(z_hbm, acc_sh.at[pl.ds(r0, npt), :])
        pltpu.sync_copy(src_hbm.at[wid, 0], si_v.at[0])
        pltpu.sync_copy(dst_hbm.at[wid, 0], di_v.at[0])
        issue_stage(1, stsems[1])
        issue_stage(2, stsems[0])
        pltpu.async_copy(y_hbm.at[si_v.at[0, 0]], rows_v.at[0], gsems[0])
        pltpu.async_copy(y_hbm.at[si_v.at[0, 1]], rows_v.at[1], gsems[1])
        plsc.subcore_barrier()

        def grp(g, _):
            bg = lax.rem(g, 3)
            bg1 = lax.rem(g + 1, 3)
            even = lax.rem(g, 2) == 0

            @pl.when(jnp.logical_and(g > 0, even))
            def _ws0():
                wait_stage(bg, stsems[0])

            @pl.when(jnp.logical_and(g > 0, jnp.logical_not(even)))
            def _ws1():
                wait_stage(bg, stsems[1])

            for t in range(NB):
                b = t % NB
                jb = (t + 2) % NB
                if t < 2:
                    @pl.when(g > 0)
                    def _wsct():
                        wait_scatter(jb)
                else:
                    wait_scatter(jb)
                if t == 2:
                    @pl.when(jnp.logical_and(g + 2 < ngrp, even))
                    def _st0():
                        issue_stage(g + 2, stsems[0])

                    @pl.when(jnp.logical_and(g + 2 < ngrp,
                                             jnp.logical_not(even)))
                    def _st1():
                        issue_stage(g + 2, stsems[1])
                if t < NB - 2:
                    nxt = si_v.at[bg, t + 2]
                else:
                    nxt = si_v.at[bg1, t + 2 - NB]
                pltpu.async_copy(y_hbm.at[nxt], rows_v.at[jb], gsems[jb])
                wait_gather(b)
                pltpu.async_copy(rows_v.at[b], acc_sh.at[di_v.at[bg, t]],
                                 ssems[b], add=True)
            return 0
        lax.fori_loop(0, ngrp, grp, 0)
        wait_scatter(NB - 2)
        wait_scatter(NB - 1)
        wait_gather(0)
        wait_gather(1)
        plsc.subcore_barrier()

        pltpu.sync_copy(acc_sh.at[pl.ds(r0, npt), :],
                        out_hbm.at[c, pl.ds(r0, npt), :])

    return k(src4d, dst4d, y, zeros_h)


def _tc_scale(xw, degp):
    n = xw.shape[0]

    def body(xw_ref, degp_ref, y_ref, dinv_ref):
        deg = degp_ref[0] + degp_ref[1] + 1.0
        dinv = lax.rsqrt(deg)
        dinv_ref[...] = dinv
        y_ref[:n] = xw_ref[...] * dinv[:n]
        y_ref[n:] = jnp.zeros((NPAD - n, D), jnp.float32)

    return pl.pallas_call(
        body,
        out_shape=(
            jax.ShapeDtypeStruct((NPAD, D), jnp.float32),
            jax.ShapeDtypeStruct((NPAD, 1), jnp.float32),
        ),
    )(xw, degp)


def _tc_final(partials, y, dinv, n):
    def body(p_ref, y_ref, dinv_ref, out_ref):
        acc = p_ref[0][:n] + p_ref[1][:n] + y_ref[:n]
        out_ref[...] = acc * dinv_ref[:n]

    return pl.pallas_call(
        body,
        out_shape=jax.ShapeDtypeStruct((n, D), jnp.float32),
    )(partials, y, dinv)


def kernel(x, edge_index, initial_weight, w_ih, w_hh, b_ih, b_hh):
    n = x.shape[0]
    e = edge_index.shape[1]
    ngrp = -(-e // (NW * NB * CHUNK))
    e2 = NW * ngrp * NB * CHUNK
    assert n < NPAD and e2 >= e and ngrp % 2 == 0

    pad = e2 - e
    padi = (jnp.arange(pad, dtype=jnp.int32) % (NPAD - n)) + n
    src4 = jnp.concatenate([edge_index[0], padi]).reshape(NW, ngrp, NB, CHUNK)
    dst4 = jnp.concatenate([edge_index[1], padi]).reshape(NW, ngrp, NB, CHUNK)
    b_ih2 = b_ih.reshape(1, 3 * D)
    b_hh2 = b_hh.reshape(1, 3 * D)
    zeros_h = jnp.zeros((NPAD // NS, D), jnp.float32)

    degp = _sc_degree(dst4)
    xw = _tc_gru_xw(x, initial_weight, w_ih, w_hh, b_ih2, b_hh2)
    y, dinv = _tc_scale(xw, degp.reshape(NC, NPAD, 1))
    partials = _sc_scatter(src4, dst4, y, zeros_h)
    return _tc_final(partials, y, dinv, n)


# speedup vs baseline: 1.2754x; 1.0228x over previous
import functools

import jax
import jax.numpy as jnp
from jax import lax
from jax.experimental import pallas as pl
from jax.experimental.pallas import tpu as pltpu
from jax.experimental.pallas import tpu_sc as plsc

D = 128
NC = 2
NS = 16
NW = NC * NS
CHUNK = 80
NB = 4
NPAD = 10240


def _tc_gru_xw(x, w0, w_ih, w_hh, b_ih2, b_hh2):
    def body(x_ref, w0_ref, wih_ref, whh_ref, bih_ref, bhh_ref, out_ref):
        w = w0_ref[...]
        gi = lax.dot_general(w, wih_ref[...], (((1,), (1,)), ((), ())),
                             preferred_element_type=jnp.float32) + bih_ref[...]
        gh = lax.dot_general(w, whh_ref[...], (((1,), (1,)), ((), ())),
                             preferred_element_type=jnp.float32) + bhh_ref[...]
        r = jax.nn.sigmoid(gi[:, :D] + gh[:, :D])
        z = jax.nn.sigmoid(gi[:, D:2 * D] + gh[:, D:2 * D])
        n = jnp.tanh(gi[:, 2 * D:] + r * gh[:, 2 * D:])
        wt = (1.0 - z) * n + z * w
        out_ref[...] = jnp.dot(x_ref[...], wt,
                               preferred_element_type=jnp.float32)

    return pl.pallas_call(
        body,
        out_shape=jax.ShapeDtypeStruct((x.shape[0], D), jnp.float32),
    )(x, w0, w_ih, w_hh, b_ih2, b_hh2)


HCH = 128
HGS = 8


def _sc_degree(dst4d):
    ngrp = dst4d.shape[1]
    spt = NPAD // NS
    mesh = plsc.VectorSubcoreMesh(core_axis_name="c", subcore_axis_name="s")

    @functools.partial(
        pl.kernel,
        out_type=jax.ShapeDtypeStruct((NC, 1, NPAD), jnp.float32),
        mesh=mesh,
        scratch_types=[
            pltpu.VMEM((HGS, HCH), jnp.int32),
            pltpu.VMEM((HCH,), jnp.float32),
            pltpu.VMEM((spt,), jnp.float32),
            pltpu.VMEM_SHARED((NPAD,), jnp.float32),
            pltpu.SemaphoreType.DMA,
        ],
    )
    def k(dst_hbm, out_hbm, idx_v, ones_v, buf_v, acc_sh, sem):
        c = lax.axis_index("c")
        s = lax.axis_index("s")
        wid = s * NC + c

        def fill_zero(i, _):
            buf_v[pl.ds(i * 16, 16)] = jnp.zeros((16,), jnp.float32)
            return 0
        lax.fori_loop(0, spt // 16, fill_zero, 0)

        def fill_one(i, _):
            ones_v[pl.ds(i * 16, 16)] = jnp.ones((16,), jnp.float32)
            return 0
        lax.fori_loop(0, HCH // 16, fill_one, 0)

        pltpu.sync_copy(buf_v, acc_sh.at[pl.ds(s * spt, spt)])
        plsc.subcore_barrier()

        def grp(g, _):
            pltpu.sync_copy(dst_hbm.at[wid, g], idx_v)
            for t in range(HGS):
                pltpu.async_copy(ones_v, acc_sh.at[idx_v.at[t]], sem, add=True)
            for t in range(HGS):
                pltpu.make_async_copy(ones_v, acc_sh.at[idx_v.at[t]], sem).wait()
            return 0
        lax.fori_loop(0, ngrp, grp, 0)
        plsc.subcore_barrier()

        pltpu.sync_copy(acc_sh.at[pl.ds(s * spt, spt)], buf_v)
        pltpu.sync_copy(buf_v, out_hbm.at[c, 0, pl.ds(s * spt, spt)])

    return k(dst4d)


def _sc_scatter(src4d, dst4d, y, zeros_h):
    ngrp = src4d.shape[1]
    npt = NPAD // NS
    mesh = plsc.VectorSubcoreMesh(core_axis_name="c", subcore_axis_name="s")

    @functools.partial(
        pl.kernel,
        out_type=jax.ShapeDtypeStruct((NC, NPAD, D), jnp.float32),
        mesh=mesh,
        scratch_types=[
            pltpu.VMEM((3, NB, CHUNK), jnp.int32),
            pltpu.VMEM((3, NB, CHUNK), jnp.int32),
            pltpu.VMEM((NB, CHUNK, D), jnp.float32),
            pltpu.VMEM_SHARED((NPAD, D), jnp.float32),
            [pltpu.SemaphoreType.DMA] * NB,
            [pltpu.SemaphoreType.DMA] * NB,
            [pltpu.SemaphoreType.DMA] * 2,
        ],
    )
    def k(src_hbm, dst_hbm, y_hbm, z_hbm, out_hbm,
          si_v, di_v, rows_v, acc_sh, gsems, ssems, stsems):
        c = lax.axis_index("c")
        s = lax.axis_index("s")
        wid = s * NC + c
        r0 = s * npt

        def issue_stage(m, sem):
            b = lax.rem(m, 3)
            pltpu.async_copy(src_hbm.at[wid, m], si_v.at[b], sem)
            pltpu.async_copy(dst_hbm.at[wid, m], di_v.at[b], sem)

        def wait_stage(bg, sem):
            pltpu.make_async_copy(src_hbm.at[wid, 0], si_v.at[bg], sem).wait()
            pltpu.make_async_copy(dst_hbm.at[wid, 0], di_v.at[bg], sem).wait()

        def wait_gather(b):
            pltpu.make_async_copy(
                y_hbm.at[si_v.at[0, 0]], rows_v.at[b], gsems[b]).wait()

        def wait_scatter(b):
            pltpu.make_async_copy(
                rows_v.at[b], acc_sh.at[di_v.at[0, 0]], ssems[b]).wait()

        pltpu.sync_copy(z_hbm, acc_sh.at[pl.ds(r0, npt), :])
        pltpu.sync_copy(src_hbm.at[wid, 0], si_v.at[0])
        pltpu.sync_copy(dst_hbm.at[wid, 0], di_v.at[0])
        issue_stage(1, stsems[1])
        issue_stage(2, stsems[0])
        pltpu.async_copy(y_hbm.at[si_v.at[0, 0]], rows_v.at[0], gsems[0])
        pltpu.async_copy(y_hbm.at[si_v.at[0, 1]], rows_v.at[1], gsems[1])
        plsc.subcore_barrier()

        def grp(g, _):
            bg = lax.rem(g, 3)
            bg1 = lax.rem(g + 1, 3)
            even = lax.rem(g, 2) == 0

            @pl.when(jnp.logical_and(g > 0, even))
            def _ws0():
                wait_stage(bg, stsems[0])

            @pl.when(jnp.logical_and(g > 0, jnp.logical_not(even)))
            def _ws1():
                wait_stage(bg, stsems[1])

            for t in range(NB):
                b = t % NB
                jb = (t + 2) % NB
                if t < 2:
                    @pl.when(g > 0)
                    def _wsct():
                        wait_scatter(jb)
                else:
                    wait_scatter(jb)
                if t == 2:
                    @pl.when(jnp.logical_and(g + 2 < ngrp, even))
                    def _st0():
                        issue_stage(g + 2, stsems[0])

                    @pl.when(jnp.logical_and(g + 2 < ngrp,
                                             jnp.logical_not(even)))
                    def _st1():
                        issue_stage(g + 2, stsems[1])
                if t < NB - 2:
                    nxt = si_v.at[bg, t + 2]
                else:
                    nxt = si_v.at[bg1, t + 2 - NB]
                pltpu.async_copy(y_hbm.at[nxt], rows_v.at[jb], gsems[jb])
                wait_gather(b)
                pltpu.async_copy(rows_v.at[b], acc_sh.at[di_v.at[bg, t]],
                                 ssems[b], add=True)
            return 0
        lax.fori_loop(0, ngrp, grp, 0)
        wait_scatter(NB - 2)
        wait_scatter(NB - 1)
        wait_gather(0)
        wait_gather(1)
        plsc.subcore_barrier()

        pltpu.sync_copy(acc_sh.at[pl.ds(r0, npt), :],
                        out_hbm.at[c, pl.ds(r0, npt), :])

    return k(src4d, dst4d, y, zeros_h)


def _tc_scale(xw, degp):
    n = xw.shape[0]

    def body(xw_ref, degp_ref, y_ref, dinv_ref):
        deg = degp_ref[0] + degp_ref[1] + 1.0
        dinv = lax.rsqrt(deg)
        dinv_ref[...] = dinv
        y_ref[:n] = xw_ref[...] * dinv[:n]
        y_ref[n:] = jnp.zeros((NPAD - n, D), jnp.float32)

    return pl.pallas_call(
        body,
        out_shape=(
            jax.ShapeDtypeStruct((NPAD, D), jnp.float32),
            jax.ShapeDtypeStruct((NPAD, 1), jnp.float32),
        ),
    )(xw, degp)


def _tc_final(partials, y, dinv, n):
    def body(p_ref, y_ref, dinv_ref, out_ref):
        acc = p_ref[0][:n] + p_ref[1][:n] + y_ref[:n]
        out_ref[...] = acc * dinv_ref[:n]

    return pl.pallas_call(
        body,
        out_shape=jax.ShapeDtypeStruct((n, D), jnp.float32),
    )(partials, y, dinv)


def kernel(x, edge_index, initial_weight, w_ih, w_hh, b_ih, b_hh):
    n = x.shape[0]
    e = edge_index.shape[1]
    ngrp = -(-e // (NW * NB * CHUNK))
    e2 = NW * ngrp * NB * CHUNK
    assert n < NPAD and e2 >= e and ngrp % 2 == 0

    assert e2 % (NW * HGS * HCH) == 0
    pad = e2 - e
    padi = (jnp.arange(pad, dtype=jnp.int32) % (NPAD - n)) + n
    src_all = jnp.concatenate([edge_index[0], padi])
    dst_all = jnp.concatenate([edge_index[1], padi])
    src4 = src_all.reshape(NW, ngrp, NB, CHUNK)
    dst4 = dst_all.reshape(NW, ngrp, NB, CHUNK)
    dst4h = dst_all.reshape(NW, e2 // (NW * HGS * HCH), HGS, HCH)
    b_ih2 = b_ih.reshape(1, 3 * D)
    b_hh2 = b_hh.reshape(1, 3 * D)
    zeros_h = jnp.zeros((NPAD // NS, D), jnp.float32)

    degp = _sc_degree(dst4h)
    xw = _tc_gru_xw(x, initial_weight, w_ih, w_hh, b_ih2, b_hh2)
    y, dinv = _tc_scale(xw, degp.reshape(NC, NPAD, 1))
    partials = _sc_scatter(src4, dst4, y, zeros_h)
    return _tc_final(partials, y, dinv, n)
